# baseline (speedup 1.0000x reference)
import functools

import jax
import jax.numpy as jnp
from jax import lax
from jax.experimental import pallas as pl
from jax.experimental.pallas import tpu as pltpu
from jax.experimental.pallas import tpu_sc as plsc

_N = 10000
_E = 320000
_DIN = 128
_D = 64
_DE = 16
_G = 128
_H = 128

_NC = 2
_NS = 16
_NW = _NC * _NS
_EPT = _E // _NW
_K = 40
_NCH = _EPT // _K
_RPT = _N // _NS

_LOG1P = (
    2.554673020349618e-07,
    0.9999670809438443,
    -0.49928504912226557,
    0.32722571497202635,
    -0.22316586411450423,
    0.130833427976782,
    -0.05243753706207599,
    0.01000928961639147,
)


def _sc_edge_body(a_hbm, b_hbm, c_hbm, src_hbm, dst_hbm, out_hbm,
                  agg_sh, src_v, dst_v,
                  a0, a1, b0, b1, c0, c1, m0, m1,
                  zrow_v, gs0, gs1, ss0, ss1):
    cid = lax.axis_index("c")
    sid = lax.axis_index("s")
    wid = cid * _NS + sid
    ar = (a0, a1)
    br = (b0, b1)
    cr = (c0, c1)
    mr = (m0, m1)
    gs = (gs0, gs1)
    ss = (ss0, ss1)

    def _zero(i, carry):
        zv = jnp.zeros((16,), jnp.float32)
        for j in range(4):
            zrow_v[i, pl.ds(j * 16, 16)] = zv
        return carry
    lax.fori_loop(0, _RPT // 5, _zero, 0)
    base_row = sid * _RPT
    for t in range(5):
        pltpu.sync_copy(
            zrow_v, agg_sh.at[pl.ds(base_row + t * (_RPT // 5), _RPT // 5)])

    pltpu.sync_copy(src_hbm.at[wid], src_v)
    pltpu.sync_copy(dst_hbm.at[wid], dst_v)
    plsc.subcore_barrier()

    cbase = wid * _EPT

    def _issue(i, q):
        pltpu.async_copy(a_hbm.at[src_v.at[i]], ar[q], gs[q])
        pltpu.async_copy(b_hbm.at[dst_v.at[i]], br[q], gs[q])
        pltpu.async_copy(c_hbm.at[pl.ds(cbase + i * _K, _K)], cr[q], gs[q])

    def _drain_gather(q):
        pltpu.make_async_copy(a_hbm.at[pl.ds(0, _K)], ar[q], gs[q]).wait()
        pltpu.make_async_copy(b_hbm.at[pl.ds(0, _K)], br[q], gs[q]).wait()
        pltpu.make_async_copy(c_hbm.at[pl.ds(0, _K)], cr[q], gs[q]).wait()

    def _drain_scatter(q):
        pltpu.make_async_copy(
            out_hbm.at[cid, sid, pl.ds(0, _K)], mr[q], ss[q]).wait()

    def _compute(q):
        aq, bq, cq, mq = ar[q], br[q], cr[q], mr[q]

        @plsc.parallel_loop(0, _K, 1, unroll=2)
        def _edge(k):
            msk = jnp.int32(-65536)
            for j in range(4):
                wa = aq[k, pl.ds(j * 16, 16)]
                wb = bq[k, pl.ds(j * 16, 16)]
                wc = cq[k, pl.ds(j * 16, 16)]
                lf = (lax.bitcast_convert_type(wa << 16, jnp.float32)
                      + lax.bitcast_convert_type(wb << 16, jnp.float32)
                      + lax.bitcast_convert_type(wc << 16, jnp.float32))
                ls = (lax.bitcast_convert_type(wa & msk, jnp.float32)
                      + lax.bitcast_convert_type(wb & msk, jnp.float32)
                      + lax.bitcast_convert_type(wc & msk, jnp.float32))
                sig = 1.0 / (1.0 + jnp.exp(-lf))
                t = jnp.exp(-jnp.abs(ls))
                p = jnp.full((16,), _LOG1P[7], jnp.float32)
                for c in _LOG1P[6::-1]:
                    p = p * t + c
                sp = jnp.maximum(ls, 0.0) + p
                mq[k, pl.ds(j * 16, 16)] = sig * sp

    for q in range(2):
        _issue(q, q)

    def _pair(ii, carry):
        for q in range(2):
            i = ii * 2 + q
            _drain_gather(q)

            @pl.when(i >= 2)
            def _():
                _drain_scatter(q)
            _compute(q)
            pltpu.async_copy(mr[q], agg_sh.at[src_v.at[i]], ss[q], add=True)

            @pl.when(i + 2 < _NCH)
            def _():
                _issue(i + 2, q)
        return carry
    lax.fori_loop(0, _NCH // 2, _pair, 0)

    for q in range(2):
        _drain_scatter(q)
    plsc.subcore_barrier()
    pltpu.sync_copy(agg_sh.at[pl.ds(base_row, _RPT)], out_hbm.at[cid, sid])


_sc_edge = pl.kernel(
    _sc_edge_body,
    out_type=jax.ShapeDtypeStruct((_NC, _NS, _RPT, _D), jnp.float32),
    mesh=plsc.VectorSubcoreMesh(core_axis_name="c", subcore_axis_name="s",
                                num_cores=_NC, num_subcores=_NS),
    scratch_types=[
        pltpu.VMEM_SHARED((_N, _D), jnp.float32),
        pltpu.VMEM((_NCH, _K), jnp.int32),
        pltpu.VMEM((_NCH, _K), jnp.int32),
        pltpu.VMEM((_K, _D), jnp.int32),
        pltpu.VMEM((_K, _D), jnp.int32),
        pltpu.VMEM((_K, _D), jnp.int32),
        pltpu.VMEM((_K, _D), jnp.int32),
        pltpu.VMEM((_K, _D), jnp.int32),
        pltpu.VMEM((_K, _D), jnp.int32),
        pltpu.VMEM((_K, _D), jnp.float32),
        pltpu.VMEM((_K, _D), jnp.float32),
        pltpu.VMEM((_RPT // 5, _D), jnp.float32),
        pltpu.SemaphoreType.DMA,
        pltpu.SemaphoreType.DMA,
        pltpu.SemaphoreType.DMA,
        pltpu.SemaphoreType.DMA,
    ],
    compiler_params=pltpu.CompilerParams(use_tc_tiling_on_sc=False),
)


def _pack_bf16_pair(fs):
    f = fs[:, :_D].astype(jnp.bfloat16).astype(jnp.float32)
    s = fs[:, _D:].astype(jnp.bfloat16).astype(jnp.float32)
    fi = lax.bitcast_convert_type(f, jnp.int32)
    si = lax.bitcast_convert_type(s, jnp.int32)
    return si | lax.shift_right_logical(fi, 16)


def _embed_body(x_ref, we_ref, be_ref, wi_ref, wj_ref, h_ref, a_ref, b_ref):
    h = jnp.dot(x_ref[...], we_ref[...],
                preferred_element_type=jnp.float32) + be_ref[...]
    h_ref[...] = h
    a_ref[...] = _pack_bf16_pair(
        jnp.dot(h, wi_ref[...], preferred_element_type=jnp.float32))
    b_ref[...] = _pack_bf16_pair(
        jnp.dot(h, wj_ref[...], preferred_element_type=jnp.float32))


def _embed_call(x, we, be, wi, wj):
    blk = 1000
    grid = _N // blk
    return pl.pallas_call(
        _embed_body,
        grid=(grid,),
        in_specs=[
            pl.BlockSpec((blk, _DIN), lambda i: (i, 0)),
            pl.BlockSpec((_DIN, _D), lambda i: (0, 0)),
            pl.BlockSpec((1, _D), lambda i: (0, 0)),
            pl.BlockSpec((_D, 2 * _D), lambda i: (0, 0)),
            pl.BlockSpec((_D, 2 * _D), lambda i: (0, 0)),
        ],
        out_specs=[
            pl.BlockSpec((blk, _D), lambda i: (i, 0)),
            pl.BlockSpec((blk, _D), lambda i: (i, 0)),
            pl.BlockSpec((blk, _D), lambda i: (i, 0)),
        ],
        out_shape=[
            jax.ShapeDtypeStruct((_N, _D), jnp.float32),
            jax.ShapeDtypeStruct((_N, _D), jnp.int32),
            jax.ShapeDtypeStruct((_N, _D), jnp.int32),
        ],
    )(x, we, be, wi, wj)


def _cconst_body(ea_ref, w_ref, b_ref, *out_refs):
    for l in range(3):
        out_refs[l][...] = _pack_bf16_pair(jnp.dot(
            ea_ref[...], w_ref[l],
            preferred_element_type=jnp.float32) + b_ref[l])


def _cconst_call(edge_attr, w_stack, b_stack):
    blk = 10000
    grid = _E // blk
    return pl.pallas_call(
        _cconst_body,
        grid=(grid,),
        in_specs=[
            pl.BlockSpec((blk, _DE), lambda i: (i, 0)),
            pl.BlockSpec((3, _DE, 2 * _D), lambda i: (0, 0, 0)),
            pl.BlockSpec((3, 1, 2 * _D), lambda i: (0, 0, 0)),
        ],
        out_specs=[pl.BlockSpec((blk, _D), lambda i: (i, 0))] * 3,
        out_shape=[jax.ShapeDtypeStruct((_E, _D), jnp.int32)] * 3,
    )(edge_attr, w_stack, b_stack)


def _bn_softplus(agg2, h, gam, bet):
    agg = agg2[0] + agg2[1]
    mean = jnp.mean(agg, axis=0, keepdims=True)
    var = jnp.mean((agg - mean) ** 2, axis=0, keepdims=True)
    bn = (agg - mean) * lax.rsqrt(var + 1e-5) * gam + bet
    return jax.nn.softplus(bn + h)


def _post_body(agg2_ref, h_ref, gam_ref, bet_ref, wi_ref, wj_ref,
               hn_ref, a_ref, b_ref):
    hn = _bn_softplus(agg2_ref[...], h_ref[...], gam_ref[...], bet_ref[...])
    hn_ref[...] = hn
    a_ref[...] = _pack_bf16_pair(
        jnp.dot(hn, wi_ref[...], preferred_element_type=jnp.float32))
    b_ref[...] = _pack_bf16_pair(
        jnp.dot(hn, wj_ref[...], preferred_element_type=jnp.float32))


def _post_call(agg2, h, gam, bet, wi, wj):
    return pl.pallas_call(
        _post_body,
        out_shape=[
            jax.ShapeDtypeStruct((_N, _D), jnp.float32),
            jax.ShapeDtypeStruct((_N, _D), jnp.int32),
            jax.ShapeDtypeStruct((_N, _D), jnp.int32),
        ],
    )(agg2, h, gam, bet, wi, wj)


def _final_body(agg2_ref, h_ref, gam_ref, bet_ref, brow_ref, bcol_ref,
                wfc_ref, bfc_ref, wout_ref, bout_ref,
                out_ref, ci_ref, mx_scr):
    hn = _bn_softplus(agg2_ref[...], h_ref[...], gam_ref[...], bet_ref[...])
    gids = lax.broadcasted_iota(jnp.int32, (_G, _N), 0)
    oneh = (brow_ref[...] == gids).astype(jnp.float32)
    counts = jnp.sum(oneh, axis=1, keepdims=True)
    sums = jnp.dot(oneh, hn, preferred_element_type=jnp.float32)
    mean_pool = sums / jnp.maximum(counts, 1.0)

    def _seg(g, carry):
        m = bcol_ref[...] == g
        vals = jnp.where(m, hn, -jnp.inf)
        mx_scr[pl.ds(g, 1), :] = jnp.max(vals, axis=0, keepdims=True)
        return carry
    lax.fori_loop(0, _G, _seg, 0)

    ci = jnp.concatenate([mean_pool, mx_scr[...]], axis=1)
    ci_ref[...] = ci
    c = jnp.dot(jax.nn.softplus(ci), wfc_ref[...],
                preferred_element_type=jnp.float32) + bfc_ref[...]
    c = jax.nn.softplus(c)
    out_ref[...] = jnp.dot(c, wout_ref[...],
                           preferred_element_type=jnp.float32) + bout_ref[...]


def _final_call(agg2, h, gam, bet, brow, bcol, wfc, bfc, wout, bout):
    return pl.pallas_call(
        _final_body,
        out_shape=[
            jax.ShapeDtypeStruct((_G, 1), jnp.float32),
            jax.ShapeDtypeStruct((_G, 2 * _D), jnp.float32),
        ],
        scratch_shapes=[pltpu.VMEM((_G, _D), jnp.float32)],
    )(agg2, h, gam, bet, brow, bcol, wfc, bfc, wout, bout)


def kernel(x, edge_index, edge_attr, batch, params):
    convs = params["convs"]
    src = edge_index[0].reshape(_NW, _NCH, _K)
    dst = edge_index[1].reshape(_NW, _NCH, _K)
    wi = [jnp.concatenate([c["Wf"][:_D], c["Ws"][:_D]], axis=1)
          for c in convs]
    wj = [jnp.concatenate([c["Wf"][_D:2 * _D], c["Ws"][_D:2 * _D]], axis=1)
          for c in convs]
    we_stack = jnp.stack(
        [jnp.concatenate([c["Wf"][2 * _D:], c["Ws"][2 * _D:]], axis=1)
         for c in convs])
    be_stack = jnp.stack(
        [jnp.concatenate([c["bf"], c["bs"]])[None] for c in convs])

    cconst = _cconst_call(edge_attr, we_stack, be_stack)
    h, a, b = _embed_call(x, params["W_emb"], params["b_emb"][None],
                          wi[0], wj[0])
    out = ci = None
    for l in range(3):
        agg2 = _sc_edge(a, b, cconst[l], src, dst).reshape(_NC, _N, _D)
        gam = convs[l]["gamma"][None]
        bet = convs[l]["beta"][None]
        if l < 2:
            h, a, b = _post_call(agg2, h, gam, bet, wi[l + 1], wj[l + 1])
        else:
            out, ci = _final_call(
                agg2, h, gam, bet, batch[None], batch[:, None],
                params["W_fc"], params["b_fc"][None],
                params["W_out"], params["b_out"][None])
    return out, ci


# speedup vs baseline: 1.1260x; 1.1260x over previous
import functools

import jax
import jax.numpy as jnp
from jax import lax
from jax.experimental import pallas as pl
from jax.experimental.pallas import tpu as pltpu
from jax.experimental.pallas import tpu_sc as plsc

_N = 10000
_E = 320000
_DIN = 128
_D = 64
_DE = 16
_G = 128
_H = 128

_NC = 2
_NS = 16
_NW = _NC * _NS
_EPT = _E // _NW
_K = 40
_NCH = _EPT // _K
_RPT = _N // _NS

_LOG1P = (
    2.554673020349618e-07,
    0.9999670809438443,
    -0.49928504912226557,
    0.32722571497202635,
    -0.22316586411450423,
    0.130833427976782,
    -0.05243753706207599,
    0.01000928961639147,
)


_NB = 5


def _sc_edge_body(a_hbm, b_hbm, c_hbm, src_hbm, dst_hbm, out_hbm,
                  agg_sh, src_v, dst_v, *bufs):
    cid = lax.axis_index("c")
    sid = lax.axis_index("s")
    wid = cid * _NS + sid
    ar = bufs[0:_NB]
    br = bufs[_NB:2 * _NB]
    cr = bufs[2 * _NB:3 * _NB]
    mr = bufs[3 * _NB:4 * _NB]
    zrow_v = bufs[4 * _NB]
    gs = bufs[4 * _NB + 1:5 * _NB + 1]
    ss = bufs[5 * _NB + 1:6 * _NB + 1]

    def _zero(i, carry):
        zv = jnp.zeros((16,), jnp.float32)
        for j in range(4):
            zrow_v[i, pl.ds(j * 16, 16)] = zv
        return carry
    lax.fori_loop(0, _RPT // 5, _zero, 0)
    base_row = sid * _RPT
    for t in range(5):
        pltpu.sync_copy(
            zrow_v, agg_sh.at[pl.ds(base_row + t * (_RPT // 5), _RPT // 5)])

    pltpu.sync_copy(src_hbm.at[wid], src_v)
    pltpu.sync_copy(dst_hbm.at[wid], dst_v)
    plsc.subcore_barrier()

    cbase = wid * _EPT

    def _issue(i, q):
        pltpu.async_copy(a_hbm.at[src_v.at[i]], ar[q], gs[q])
        pltpu.async_copy(b_hbm.at[dst_v.at[i]], br[q], gs[q])
        pltpu.async_copy(c_hbm.at[pl.ds(cbase + i * _K, _K)], cr[q], gs[q])

    def _drain_gather(q):
        pltpu.make_async_copy(a_hbm.at[pl.ds(0, _K)], ar[q], gs[q]).wait()
        pltpu.make_async_copy(b_hbm.at[pl.ds(0, _K)], br[q], gs[q]).wait()
        pltpu.make_async_copy(c_hbm.at[pl.ds(0, _K)], cr[q], gs[q]).wait()

    def _drain_scatter(q):
        pltpu.make_async_copy(
            out_hbm.at[cid, sid, pl.ds(0, _K)], mr[q], ss[q]).wait()

    def _compute(q):
        aq, bq, cq, mq = ar[q], br[q], cr[q], mr[q]

        @plsc.parallel_loop(0, _K, 1, unroll=2)
        def _edge(k):
            msk = jnp.int32(-65536)
            for j in range(4):
                wa = plsc.bitcast(aq[k, pl.ds(j * 16, 16)], jnp.bfloat16)
                wb = plsc.bitcast(bq[k, pl.ds(j * 16, 16)], jnp.bfloat16)
                wc = plsc.bitcast(cq[k, pl.ds(j * 16, 16)], jnp.bfloat16)
                w = plsc.bitcast(wa + wb + wc, jnp.int32)
                lf = lax.bitcast_convert_type(w << 16, jnp.float32)
                ls = lax.bitcast_convert_type(w & msk, jnp.float32)
                sig = 1.0 / (1.0 + jnp.exp(-lf))
                t = jnp.exp(-jnp.abs(ls))
                p = jnp.full((16,), _LOG1P[7], jnp.float32)
                for c in _LOG1P[6::-1]:
                    p = p * t + c
                sp = jnp.maximum(ls, 0.0) + p
                mq[k, pl.ds(j * 16, 16)] = sig * sp

    for q in range(_NB):
        _issue(q, q)

    def _ring(ii, carry):
        for q in range(_NB):
            i = ii * _NB + q
            _drain_gather(q)

            @pl.when(i >= _NB)
            def _():
                _drain_scatter(q)
            _compute(q)
            pltpu.async_copy(mr[q], agg_sh.at[src_v.at[i]], ss[q], add=True)

            @pl.when(i + _NB < _NCH)
            def _():
                _issue(i + _NB, q)
        return carry
    lax.fori_loop(0, _NCH // _NB, _ring, 0)

    for q in range(_NB):
        _drain_scatter(q)
    plsc.subcore_barrier()
    pltpu.sync_copy(agg_sh.at[pl.ds(base_row, _RPT)], out_hbm.at[cid, sid])


_sc_edge = pl.kernel(
    _sc_edge_body,
    out_type=jax.ShapeDtypeStruct((_NC, _NS, _RPT, _D), jnp.float32),
    mesh=plsc.VectorSubcoreMesh(core_axis_name="c", subcore_axis_name="s",
                                num_cores=_NC, num_subcores=_NS),
    scratch_types=(
        [pltpu.VMEM_SHARED((_N, _D), jnp.float32)]
        + [pltpu.VMEM((_NCH, _K), jnp.int32)] * 2
        + [pltpu.VMEM((_K, _D), jnp.int32)] * (3 * _NB)
        + [pltpu.VMEM((_K, _D), jnp.float32)] * _NB
        + [pltpu.VMEM((_RPT // 5, _D), jnp.float32)]
        + [pltpu.SemaphoreType.DMA] * (2 * _NB)
    ),
    compiler_params=pltpu.CompilerParams(use_tc_tiling_on_sc=False,
                                         needs_layout_passes=False),
)


def _pack_bf16_pair(fs):
    f = fs[:, :_D].astype(jnp.bfloat16).astype(jnp.float32)
    s = fs[:, _D:].astype(jnp.bfloat16).astype(jnp.float32)
    fi = lax.bitcast_convert_type(f, jnp.int32)
    si = lax.bitcast_convert_type(s, jnp.int32)
    return si | lax.shift_right_logical(fi, 16)


def _embed_body(x_ref, we_ref, be_ref, wi_ref, wj_ref, h_ref, a_ref, b_ref):
    h = jnp.dot(x_ref[...], we_ref[...],
                preferred_element_type=jnp.float32) + be_ref[...]
    h_ref[...] = h
    a_ref[...] = _pack_bf16_pair(
        jnp.dot(h, wi_ref[...], preferred_element_type=jnp.float32))
    b_ref[...] = _pack_bf16_pair(
        jnp.dot(h, wj_ref[...], preferred_element_type=jnp.float32))


def _embed_call(x, we, be, wi, wj):
    blk = 1000
    grid = _N // blk
    return pl.pallas_call(
        _embed_body,
        grid=(grid,),
        in_specs=[
            pl.BlockSpec((blk, _DIN), lambda i: (i, 0)),
            pl.BlockSpec((_DIN, _D), lambda i: (0, 0)),
            pl.BlockSpec((1, _D), lambda i: (0, 0)),
            pl.BlockSpec((_D, 2 * _D), lambda i: (0, 0)),
            pl.BlockSpec((_D, 2 * _D), lambda i: (0, 0)),
        ],
        out_specs=[
            pl.BlockSpec((blk, _D), lambda i: (i, 0)),
            pl.BlockSpec((blk, _D), lambda i: (i, 0)),
            pl.BlockSpec((blk, _D), lambda i: (i, 0)),
        ],
        out_shape=[
            jax.ShapeDtypeStruct((_N, _D), jnp.float32),
            jax.ShapeDtypeStruct((_N, _D), jnp.int32),
            jax.ShapeDtypeStruct((_N, _D), jnp.int32),
        ],
    )(x, we, be, wi, wj)


def _cconst_body(ea_ref, w_ref, b_ref, *out_refs):
    for l in range(3):
        out_refs[l][...] = _pack_bf16_pair(jnp.dot(
            ea_ref[...], w_ref[l],
            preferred_element_type=jnp.float32) + b_ref[l])


def _cconst_call(edge_attr, w_stack, b_stack):
    blk = 10000
    grid = _E // blk
    return pl.pallas_call(
        _cconst_body,
        grid=(grid,),
        in_specs=[
            pl.BlockSpec((blk, _DE), lambda i: (i, 0)),
            pl.BlockSpec((3, _DE, 2 * _D), lambda i: (0, 0, 0)),
            pl.BlockSpec((3, 1, 2 * _D), lambda i: (0, 0, 0)),
        ],
        out_specs=[pl.BlockSpec((blk, _D), lambda i: (i, 0))] * 3,
        out_shape=[jax.ShapeDtypeStruct((_E, _D), jnp.int32)] * 3,
    )(edge_attr, w_stack, b_stack)


def _bn_softplus(agg2, h, gam, bet):
    agg = agg2[0] + agg2[1]
    mean = jnp.mean(agg, axis=0, keepdims=True)
    var = jnp.mean((agg - mean) ** 2, axis=0, keepdims=True)
    bn = (agg - mean) * lax.rsqrt(var + 1e-5) * gam + bet
    return jax.nn.softplus(bn + h)


def _post_body(agg2_ref, h_ref, gam_ref, bet_ref, wi_ref, wj_ref,
               hn_ref, a_ref, b_ref):
    hn = _bn_softplus(agg2_ref[...], h_ref[...], gam_ref[...], bet_ref[...])
    hn_ref[...] = hn
    a_ref[...] = _pack_bf16_pair(
        jnp.dot(hn, wi_ref[...], preferred_element_type=jnp.float32))
    b_ref[...] = _pack_bf16_pair(
        jnp.dot(hn, wj_ref[...], preferred_element_type=jnp.float32))


def _post_call(agg2, h, gam, bet, wi, wj):
    return pl.pallas_call(
        _post_body,
        out_shape=[
            jax.ShapeDtypeStruct((_N, _D), jnp.float32),
            jax.ShapeDtypeStruct((_N, _D), jnp.int32),
            jax.ShapeDtypeStruct((_N, _D), jnp.int32),
        ],
    )(agg2, h, gam, bet, wi, wj)


def _final_body(agg2_ref, h_ref, gam_ref, bet_ref, brow_ref, bcol_ref,
                wfc_ref, bfc_ref, wout_ref, bout_ref,
                out_ref, ci_ref, mx_scr):
    hn = _bn_softplus(agg2_ref[...], h_ref[...], gam_ref[...], bet_ref[...])
    gids = lax.broadcasted_iota(jnp.int32, (_G, _N), 0)
    oneh = (brow_ref[...] == gids).astype(jnp.float32)
    counts = jnp.sum(oneh, axis=1, keepdims=True)
    sums = jnp.dot(oneh, hn, preferred_element_type=jnp.float32)
    mean_pool = sums / jnp.maximum(counts, 1.0)

    def _seg(g, carry):
        m = bcol_ref[...] == g
        vals = jnp.where(m, hn, -jnp.inf)
        mx_scr[pl.ds(g, 1), :] = jnp.max(vals, axis=0, keepdims=True)
        return carry
    lax.fori_loop(0, _G, _seg, 0)

    ci = jnp.concatenate([mean_pool, mx_scr[...]], axis=1)
    ci_ref[...] = ci
    c = jnp.dot(jax.nn.softplus(ci), wfc_ref[...],
                preferred_element_type=jnp.float32) + bfc_ref[...]
    c = jax.nn.softplus(c)
    out_ref[...] = jnp.dot(c, wout_ref[...],
                           preferred_element_type=jnp.float32) + bout_ref[...]


def _final_call(agg2, h, gam, bet, brow, bcol, wfc, bfc, wout, bout):
    return pl.pallas_call(
        _final_body,
        out_shape=[
            jax.ShapeDtypeStruct((_G, 1), jnp.float32),
            jax.ShapeDtypeStruct((_G, 2 * _D), jnp.float32),
        ],
        scratch_shapes=[pltpu.VMEM((_G, _D), jnp.float32)],
    )(agg2, h, gam, bet, brow, bcol, wfc, bfc, wout, bout)


def kernel(x, edge_index, edge_attr, batch, params):
    convs = params["convs"]
    src = edge_index[0].reshape(_NW, _NCH, _K)
    dst = edge_index[1].reshape(_NW, _NCH, _K)
    wi = [jnp.concatenate([c["Wf"][:_D], c["Ws"][:_D]], axis=1)
          for c in convs]
    wj = [jnp.concatenate([c["Wf"][_D:2 * _D], c["Ws"][_D:2 * _D]], axis=1)
          for c in convs]
    we_stack = jnp.stack(
        [jnp.concatenate([c["Wf"][2 * _D:], c["Ws"][2 * _D:]], axis=1)
         for c in convs])
    be_stack = jnp.stack(
        [jnp.concatenate([c["bf"], c["bs"]])[None] for c in convs])

    cconst = _cconst_call(edge_attr, we_stack, be_stack)
    h, a, b = _embed_call(x, params["W_emb"], params["b_emb"][None],
                          wi[0], wj[0])
    out = ci = None
    for l in range(3):
        agg2 = _sc_edge(a, b, cconst[l], src, dst).reshape(_NC, _N, _D)
        gam = convs[l]["gamma"][None]
        bet = convs[l]["beta"][None]
        if l < 2:
            h, a, b = _post_call(agg2, h, gam, bet, wi[l + 1], wj[l + 1])
        else:
            out, ci = _final_call(
                agg2, h, gam, bet, batch[None], batch[:, None],
                params["W_fc"], params["b_fc"][None],
                params["W_out"], params["b_out"][None])
    return out, ci


# speedup vs baseline: 1.4601x; 1.2967x over previous
import functools

import jax
import jax.numpy as jnp
from jax import lax
from jax.experimental import pallas as pl
from jax.experimental.pallas import tpu as pltpu
from jax.experimental.pallas import tpu_sc as plsc

_N = 10000
_E = 320000
_DIN = 128
_D = 64
_DE = 16
_G = 128
_H = 128

_NC = 2
_NS = 16
_NW = _NC * _NS
_EPT = _E // _NW
_K = 40
_NCH = _EPT // _K
_RPT = _N // _NS

_LOG1P = (
    2.554673020349618e-07,
    0.9999670809438443,
    -0.49928504912226557,
    0.32722571497202635,
    -0.22316586411450423,
    0.130833427976782,
    -0.05243753706207599,
    0.01000928961639147,
)


_NB = 2


def _sc_edge_body(a_hbm, b_hbm, c_hbm, src_hbm, dst_hbm, out_hbm,
                  agg_sh, src_v, dst_v, *bufs):
    cid = lax.axis_index("c")
    sid = lax.axis_index("s")
    wid = cid * _NS + sid
    ar = bufs[0:_NB]
    br = bufs[_NB:2 * _NB]
    cr = bufs[2 * _NB:3 * _NB]
    mr = bufs[3 * _NB:4 * _NB]
    zrow_v = bufs[4 * _NB]
    gs = bufs[4 * _NB + 1:5 * _NB + 1]
    ss = bufs[5 * _NB + 1:6 * _NB + 1]

    def _zero(i, carry):
        zv = jnp.zeros((16,), jnp.float32)
        for j in range(4):
            zrow_v[i, pl.ds(j * 16, 16)] = zv
        return carry
    lax.fori_loop(0, _RPT // 5, _zero, 0)
    base_row = sid * _RPT
    for t in range(5):
        pltpu.sync_copy(
            zrow_v, agg_sh.at[pl.ds(base_row + t * (_RPT // 5), _RPT // 5)])

    pltpu.sync_copy(src_hbm.at[wid], src_v)
    pltpu.sync_copy(dst_hbm.at[wid], dst_v)
    plsc.subcore_barrier()

    cbase = wid * _EPT

    def _issue(i, q):
        pltpu.async_copy(a_hbm.at[src_v.at[i]], ar[q], gs[q])
        pltpu.async_copy(b_hbm.at[dst_v.at[i]], br[q], gs[q])
        pltpu.async_copy(c_hbm.at[pl.ds(cbase + i * _K, _K)], cr[q], gs[q])

    def _drain_gather(q):
        pltpu.make_async_copy(a_hbm.at[pl.ds(0, _K)], ar[q], gs[q]).wait()
        pltpu.make_async_copy(b_hbm.at[pl.ds(0, _K)], br[q], gs[q]).wait()
        pltpu.make_async_copy(c_hbm.at[pl.ds(0, _K)], cr[q], gs[q]).wait()

    def _drain_scatter(q):
        pltpu.make_async_copy(
            out_hbm.at[cid, sid, pl.ds(0, _K)], mr[q], ss[q]).wait()

    def _compute(q):
        aq, bq, cq, mq = ar[q], br[q], cr[q], mr[q]

        @plsc.parallel_loop(0, _K, 1, unroll=2)
        def _edge(k):
            for j in range(4):
                lf = (aq[k, pl.ds(j * 16, 16)]
                      + bq[k, pl.ds(j * 16, 16)]
                      + cq[k, pl.ds(j * 16, 16)])
                ls = (aq[k, pl.ds(_D + j * 16, 16)]
                      + bq[k, pl.ds(_D + j * 16, 16)]
                      + cq[k, pl.ds(_D + j * 16, 16)])
                sig = 1.0 / (1.0 + jnp.exp(-lf))
                t = jnp.exp(-jnp.abs(ls))
                p = jnp.full((16,), _LOG1P[7], jnp.float32)
                for c in _LOG1P[6::-1]:
                    p = p * t + c
                sp = jnp.maximum(ls, 0.0) + p
                mq[k, pl.ds(j * 16, 16)] = sig * sp

    for q in range(_NB):
        _issue(q, q)

    def _ring(ii, carry):
        for q in range(_NB):
            i = ii * _NB + q
            _drain_gather(q)

            @pl.when(i >= _NB)
            def _():
                _drain_scatter(q)
            _compute(q)
            pltpu.async_copy(mr[q], agg_sh.at[src_v.at[i]], ss[q], add=True)

            @pl.when(i + _NB < _NCH)
            def _():
                _issue(i + _NB, q)
        return carry
    lax.fori_loop(0, _NCH // _NB, _ring, 0)

    for q in range(_NB):
        _drain_scatter(q)
    plsc.subcore_barrier()
    pltpu.sync_copy(agg_sh.at[pl.ds(base_row, _RPT)], out_hbm.at[cid, sid])


_sc_edge = pl.kernel(
    _sc_edge_body,
    out_type=jax.ShapeDtypeStruct((_NC, _NS, _RPT, _D), jnp.float32),
    mesh=plsc.VectorSubcoreMesh(core_axis_name="c", subcore_axis_name="s",
                                num_cores=_NC, num_subcores=_NS),
    scratch_types=(
        [pltpu.VMEM_SHARED((_N, _D), jnp.float32)]
        + [pltpu.VMEM((_NCH, _K), jnp.int32)] * 2
        + [pltpu.VMEM((_K, 2 * _D), jnp.float32)] * (3 * _NB)
        + [pltpu.VMEM((_K, _D), jnp.float32)] * _NB
        + [pltpu.VMEM((_RPT // 5, _D), jnp.float32)]
        + [pltpu.SemaphoreType.DMA] * (2 * _NB)
    ),
    compiler_params=pltpu.CompilerParams(use_tc_tiling_on_sc=False),
)


def _embed_body(x_ref, we_ref, be_ref, wi_ref, wj_ref, h_ref, a_ref, b_ref):
    h = jnp.dot(x_ref[...], we_ref[...],
                preferred_element_type=jnp.float32) + be_ref[...]
    h_ref[...] = h
    a_ref[...] = jnp.dot(h, wi_ref[...], preferred_element_type=jnp.float32)
    b_ref[...] = jnp.dot(h, wj_ref[...], preferred_element_type=jnp.float32)


def _embed_call(x, we, be, wi, wj):
    blk = 1000
    grid = _N // blk
    return pl.pallas_call(
        _embed_body,
        grid=(grid,),
        in_specs=[
            pl.BlockSpec((blk, _DIN), lambda i: (i, 0)),
            pl.BlockSpec((_DIN, _D), lambda i: (0, 0)),
            pl.BlockSpec((1, _D), lambda i: (0, 0)),
            pl.BlockSpec((_D, 2 * _D), lambda i: (0, 0)),
            pl.BlockSpec((_D, 2 * _D), lambda i: (0, 0)),
        ],
        out_specs=[
            pl.BlockSpec((blk, _D), lambda i: (i, 0)),
            pl.BlockSpec((blk, 2 * _D), lambda i: (i, 0)),
            pl.BlockSpec((blk, 2 * _D), lambda i: (i, 0)),
        ],
        out_shape=[
            jax.ShapeDtypeStruct((_N, _D), jnp.float32),
            jax.ShapeDtypeStruct((_N, 2 * _D), jnp.float32),
            jax.ShapeDtypeStruct((_N, 2 * _D), jnp.float32),
        ],
    )(x, we, be, wi, wj)


def _cconst_body(ea_ref, w_ref, b_ref, *out_refs):
    for l in range(3):
        out_refs[l][...] = jnp.dot(
            ea_ref[...], w_ref[l],
            preferred_element_type=jnp.float32) + b_ref[l]


def _cconst_call(edge_attr, w_stack, b_stack):
    blk = 10000
    grid = _E // blk
    return pl.pallas_call(
        _cconst_body,
        grid=(grid,),
        in_specs=[
            pl.BlockSpec((blk, _DE), lambda i: (i, 0)),
            pl.BlockSpec((3, _DE, 2 * _D), lambda i: (0, 0, 0)),
            pl.BlockSpec((3, 1, 2 * _D), lambda i: (0, 0, 0)),
        ],
        out_specs=[pl.BlockSpec((blk, 2 * _D), lambda i: (i, 0))] * 3,
        out_shape=[jax.ShapeDtypeStruct((_E, 2 * _D), jnp.float32)] * 3,
    )(edge_attr, w_stack, b_stack)


def _bn_softplus(agg2, h, gam, bet):
    agg = agg2[0] + agg2[1]
    mean = jnp.mean(agg, axis=0, keepdims=True)
    var = jnp.mean((agg - mean) ** 2, axis=0, keepdims=True)
    bn = (agg - mean) * lax.rsqrt(var + 1e-5) * gam + bet
    return jax.nn.softplus(bn + h)


def _post_body(agg2_ref, h_ref, gam_ref, bet_ref, wi_ref, wj_ref,
               hn_ref, a_ref, b_ref):
    hn = _bn_softplus(agg2_ref[...], h_ref[...], gam_ref[...], bet_ref[...])
    hn_ref[...] = hn
    a_ref[...] = jnp.dot(hn, wi_ref[...], preferred_element_type=jnp.float32)
    b_ref[...] = jnp.dot(hn, wj_ref[...], preferred_element_type=jnp.float32)


def _post_call(agg2, h, gam, bet, wi, wj):
    return pl.pallas_call(
        _post_body,
        out_shape=[
            jax.ShapeDtypeStruct((_N, _D), jnp.float32),
            jax.ShapeDtypeStruct((_N, 2 * _D), jnp.float32),
            jax.ShapeDtypeStruct((_N, 2 * _D), jnp.float32),
        ],
    )(agg2, h, gam, bet, wi, wj)


def _final_body(agg2_ref, h_ref, gam_ref, bet_ref, brow_ref, bcol_ref,
                wfc_ref, bfc_ref, wout_ref, bout_ref,
                out_ref, ci_ref, mx_scr):
    hn = _bn_softplus(agg2_ref[...], h_ref[...], gam_ref[...], bet_ref[...])
    gids = lax.broadcasted_iota(jnp.int32, (_G, _N), 0)
    oneh = (brow_ref[...] == gids).astype(jnp.float32)
    counts = jnp.sum(oneh, axis=1, keepdims=True)
    sums = jnp.dot(oneh, hn, preferred_element_type=jnp.float32)
    mean_pool = sums / jnp.maximum(counts, 1.0)

    def _seg(g, carry):
        m = bcol_ref[...] == g
        vals = jnp.where(m, hn, -jnp.inf)
        mx_scr[pl.ds(g, 1), :] = jnp.max(vals, axis=0, keepdims=True)
        return carry
    lax.fori_loop(0, _G, _seg, 0)

    ci = jnp.concatenate([mean_pool, mx_scr[...]], axis=1)
    ci_ref[...] = ci
    c = jnp.dot(jax.nn.softplus(ci), wfc_ref[...],
                preferred_element_type=jnp.float32) + bfc_ref[...]
    c = jax.nn.softplus(c)
    out_ref[...] = jnp.dot(c, wout_ref[...],
                           preferred_element_type=jnp.float32) + bout_ref[...]


def _final_call(agg2, h, gam, bet, brow, bcol, wfc, bfc, wout, bout):
    return pl.pallas_call(
        _final_body,
        out_shape=[
            jax.ShapeDtypeStruct((_G, 1), jnp.float32),
            jax.ShapeDtypeStruct((_G, 2 * _D), jnp.float32),
        ],
        scratch_shapes=[pltpu.VMEM((_G, _D), jnp.float32)],
    )(agg2, h, gam, bet, brow, bcol, wfc, bfc, wout, bout)


def kernel(x, edge_index, edge_attr, batch, params):
    convs = params["convs"]
    src = edge_index[0].reshape(_NW, _NCH, _K)
    dst = edge_index[1].reshape(_NW, _NCH, _K)
    wi = [jnp.concatenate([c["Wf"][:_D], c["Ws"][:_D]], axis=1)
          for c in convs]
    wj = [jnp.concatenate([c["Wf"][_D:2 * _D], c["Ws"][_D:2 * _D]], axis=1)
          for c in convs]
    we_stack = jnp.stack(
        [jnp.concatenate([c["Wf"][2 * _D:], c["Ws"][2 * _D:]], axis=1)
         for c in convs])
    be_stack = jnp.stack(
        [jnp.concatenate([c["bf"], c["bs"]])[None] for c in convs])

    cconst = _cconst_call(edge_attr, we_stack, be_stack)
    h, a, b = _embed_call(x, params["W_emb"], params["b_emb"][None],
                          wi[0], wj[0])
    out = ci = None
    for l in range(3):
        agg2 = _sc_edge(a, b, cconst[l], src, dst).reshape(_NC, _N, _D)
        gam = convs[l]["gamma"][None]
        bet = convs[l]["beta"][None]
        if l < 2:
            h, a, b = _post_call(agg2, h, gam, bet, wi[l + 1], wj[l + 1])
        else:
            out, ci = _final_call(
                agg2, h, gam, bet, batch[None], batch[:, None],
                params["W_fc"], params["b_fc"][None],
                params["W_out"], params["b_out"][None])
    return out, ci
